# baseline (speedup 1.0000x reference)
import functools

import jax
import jax.numpy as jnp
from jax import lax
from jax.experimental import pallas as pl
from jax.experimental.pallas import tpu as pltpu
from jax.experimental.pallas import tpu_sc as plsc

BB = 32
SC_B = 256


def _make_sc_gather(S, B, N, D, K):
    info = plsc.get_sparse_core_info()
    nw = info.num_cores * info.num_subcores
    bpw = S // nw
    mesh = plsc.VectorSubcoreMesh(core_axis_name="c", subcore_axis_name="s")

    @functools.partial(
        pl.kernel, mesh=mesh,
        out_type=jax.ShapeDtypeStruct((S * N, D), jnp.float32),
        scratch_types=[
            pltpu.VMEM((K, N), jnp.int32),
            pltpu.VMEM((K, N, D), jnp.float32),
            pltpu.VMEM_SHARED((info.num_subcores, N, D), jnp.float32),
            pltpu.VMEM((N,), jnp.int32),
            pltpu.SemaphoreType.DMA,
        ],
    )
    def sc_gather(gidx_hbm, table_hbm, ident_hbm, out_hbm,
                  idx_v, rows_v, acc_sh, ident_v, sem):
        sid = lax.axis_index("s")
        wid = sid * info.num_cores + lax.axis_index("c")
        pltpu.sync_copy(ident_hbm, ident_v)
        for t in range(bpw):
            b = wid * bpw + t
            pltpu.sync_copy(gidx_hbm.at[b], idx_v)
            cps = []
            for kk in range(K):
                cp = pltpu.make_async_copy(
                    table_hbm.at[idx_v.at[kk]], rows_v.at[kk], sem)
                cp.start()
                cps.append(cp)
            for cp in cps:
                cp.wait()
            pltpu.sync_copy(rows_v.at[0], acc_sh.at[sid].at[ident_v])
            for kk in range(1, K):
                pltpu.sync_copy(rows_v.at[kk], acc_sh.at[sid].at[ident_v],
                                add=True)
            pltpu.sync_copy(acc_sh.at[sid], out_hbm.at[pl.ds(b * N, N)])

    return sc_gather


def _tc_full_body(edges_ref, atoms_ref, w1_ref, w2_ref, aux_ref, sel_ref,
                  iota_ref, summed_ref, out1_ref, out2_ref):
    bb, n, k = edges_ref.shape
    d = atoms_ref.shape[-1]
    kcw = w1_ref.shape[-1]
    cw = kcw // k
    iota_f = iota_ref[...]
    ones_1n = jnp.ones((1, n), jnp.float32)
    b1 = aux_ref[0:1, :]
    b2 = aux_ref[1:2, :]
    jdiv = aux_ref[2:3, :]
    for i in range(bb):
        e_f = edges_ref[i]
        a_cnt = jnp.zeros((n, n), jnp.float32)
        for j in range(k):
            col = jax.lax.slice(e_f, (0, j), (n, j + 1))
            col_b = jnp.dot(col, ones_1n,
                            preferred_element_type=jnp.float32)
            a_cnt += (col_b == iota_f).astype(jnp.float32)
        summed_ref[i] = jnp.dot(a_cnt, atoms_ref[i],
                                preferred_element_type=jnp.float32)
    e_all = edges_ref[...].reshape(bb * n, k)
    mask_e = (e_all != -1.0).astype(jnp.float32)
    ones_kj = jnp.ones((k, kcw), jnp.float32)
    deg_b = jnp.dot(mask_e, ones_kj,
                    preferred_element_type=jnp.float32)
    m_sel = (deg_b == jdiv).astype(jnp.float32)
    s_all = summed_ref[...].reshape(bb * n, d)
    a_all = atoms_ref[...].reshape(bb * n, d)
    z1 = jnp.maximum(jnp.dot(s_all, w1_ref[...],
                             preferred_element_type=jnp.float32) + b1, 0.0)
    z2 = jnp.maximum(jnp.dot(a_all, w2_ref[...],
                             preferred_element_type=jnp.float32) + b2, 0.0)
    o1 = jnp.dot(z1 * m_sel, sel_ref[...],
                 preferred_element_type=jnp.float32)
    o2 = jnp.dot(z2 * m_sel, sel_ref[...],
                 preferred_element_type=jnp.float32)
    out1_ref[...] = o1.reshape(bb, n, cw)
    out2_ref[...] = o2.reshape(bb, n, cw)


def _tc_dense_body(edges_ref, atoms_ref, summed_ref, w1_ref, w2_ref, aux_ref,
                   sel_ref, dep_ref, out1_ref, out2_ref):
    del dep_ref
    bb, n, k = edges_ref.shape
    d = atoms_ref.shape[-1]
    kcw = w1_ref.shape[-1]
    cw = kcw // k
    b1 = aux_ref[0:1, :]
    b2 = aux_ref[1:2, :]
    jdiv = aux_ref[2:3, :]
    e_all = edges_ref[...].reshape(bb * n, k)
    mask_e = (e_all != -1.0).astype(jnp.float32)
    ones_kj = jnp.ones((k, kcw), jnp.float32)
    deg_b = jnp.dot(mask_e, ones_kj,
                    preferred_element_type=jnp.float32)
    m_sel = (deg_b == jdiv).astype(jnp.float32)
    s_all = summed_ref[...].reshape(bb * n, d)
    a_all = atoms_ref[...].reshape(bb * n, d)
    z1 = jnp.maximum(jnp.dot(s_all, w1_ref[...],
                             preferred_element_type=jnp.float32) + b1, 0.0)
    z2 = jnp.maximum(jnp.dot(a_all, w2_ref[...],
                             preferred_element_type=jnp.float32) + b2, 0.0)
    o1 = jnp.dot(z1 * m_sel, sel_ref[...],
                 preferred_element_type=jnp.float32)
    o2 = jnp.dot(z2 * m_sel, sel_ref[...],
                 preferred_element_type=jnp.float32)
    out1_ref[...] = o1.reshape(bb, n, cw)
    out2_ref[...] = o2.reshape(bb, n, cw)


def kernel(atoms, edges, W1, b1, W2, b2):
    B, N, D = atoms.shape
    K = edges.shape[-1]
    CW = W1.shape[-1]
    S = SC_B
    gidx = (edges[:S].transpose(0, 2, 1)
            + (jnp.arange(S, dtype=jnp.int32) * N)[:, None, None])
    table = atoms.reshape(B * N, D)
    ident = jnp.arange(N, dtype=jnp.int32)
    summed_lo = _make_sc_gather(S, B, N, D, K)(gidx, table, ident)
    summed_lo = summed_lo.reshape(S, N, D)
    w1r = W1.transpose(1, 0, 2).reshape(D, K * CW)
    w2r = W2.transpose(1, 0, 2).reshape(D, K * CW)
    aux = jnp.zeros((8, K * CW), jnp.float32)
    aux = aux.at[0].set(b1.reshape(-1)).at[1].set(b2.reshape(-1))
    aux = aux.at[2].set(jnp.repeat(jnp.arange(K, dtype=jnp.float32), CW))
    sel = jnp.tile(jnp.eye(CW, dtype=jnp.float32), (K, 1))
    edges_f = edges.astype(jnp.float32)
    iota_f = jnp.broadcast_to(jnp.arange(N, dtype=jnp.float32)[None, :],
                              (N, N))
    sb = S // BB
    summed_hi, o1_hi, o2_hi = pl.pallas_call(
        _tc_full_body,
        grid=((B - S) // BB,),
        in_specs=[
            pl.BlockSpec((BB, N, K), lambda i: (i + sb, 0, 0)),
            pl.BlockSpec((BB, N, D), lambda i: (i + sb, 0, 0)),
            pl.BlockSpec((D, K * CW), lambda i: (0, 0)),
            pl.BlockSpec((D, K * CW), lambda i: (0, 0)),
            pl.BlockSpec((8, K * CW), lambda i: (0, 0)),
            pl.BlockSpec((K * CW, CW), lambda i: (0, 0)),
            pl.BlockSpec((N, N), lambda i: (0, 0)),
        ],
        out_specs=[
            pl.BlockSpec((BB, N, D), lambda i: (i, 0, 0)),
            pl.BlockSpec((BB, N, CW), lambda i: (i, 0, 0)),
            pl.BlockSpec((BB, N, CW), lambda i: (i, 0, 0)),
        ],
        out_shape=[
            jax.ShapeDtypeStruct((B - S, N, D), jnp.float32),
            jax.ShapeDtypeStruct((B - S, N, CW), jnp.float32),
            jax.ShapeDtypeStruct((B - S, N, CW), jnp.float32),
        ],
    )(edges_f, atoms, w1r, w2r, aux, sel, iota_f)
    o1_lo, o2_lo = pl.pallas_call(
        _tc_dense_body,
        grid=(S // BB,),
        in_specs=[
            pl.BlockSpec((BB, N, K), lambda i: (i, 0, 0)),
            pl.BlockSpec((BB, N, D), lambda i: (i, 0, 0)),
            pl.BlockSpec((BB, N, D), lambda i: (i, 0, 0)),
            pl.BlockSpec((D, K * CW), lambda i: (0, 0)),
            pl.BlockSpec((D, K * CW), lambda i: (0, 0)),
            pl.BlockSpec((8, K * CW), lambda i: (0, 0)),
            pl.BlockSpec((K * CW, CW), lambda i: (0, 0)),
            pl.BlockSpec((1, N, CW), lambda i: (0, 0, 0)),
        ],
        out_specs=[
            pl.BlockSpec((BB, N, CW), lambda i: (i, 0, 0)),
            pl.BlockSpec((BB, N, CW), lambda i: (i, 0, 0)),
        ],
        out_shape=[
            jax.ShapeDtypeStruct((S, N, CW), jnp.float32),
            jax.ShapeDtypeStruct((S, N, CW), jnp.float32),
        ],
    )(edges_f, atoms, summed_lo, w1r, w2r, aux, sel, o1_hi)
    summed = jnp.concatenate([summed_lo, summed_hi], axis=0)
    o1 = jnp.concatenate([o1_lo, o1_hi], axis=0)
    o2 = jnp.concatenate([o2_lo, o2_hi], axis=0)
    return (summed, atoms, o1, o2)


# speedup vs baseline: 1.0045x; 1.0045x over previous
import functools

import jax
import jax.numpy as jnp
from jax import lax
from jax.experimental import pallas as pl
from jax.experimental.pallas import tpu as pltpu
from jax.experimental.pallas import tpu_sc as plsc

BB = 32
SC_B = 256


def _make_sc_gather(S, B, N, D, K):
    info = plsc.get_sparse_core_info()
    nw = info.num_cores * info.num_subcores
    bpw = S // nw
    mesh = plsc.VectorSubcoreMesh(core_axis_name="c", subcore_axis_name="s")

    @functools.partial(
        pl.kernel, mesh=mesh,
        out_type=jax.ShapeDtypeStruct((S * N, D), jnp.float32),
        scratch_types=[
            pltpu.VMEM((K, N), jnp.int32),
            pltpu.VMEM((K, N, D), jnp.float32),
            pltpu.VMEM_SHARED((info.num_subcores, N, D), jnp.float32),
            pltpu.VMEM((N,), jnp.int32),
            pltpu.SemaphoreType.DMA,
        ],
    )
    def sc_gather(gidx_hbm, table_hbm, ident_hbm, out_hbm,
                  idx_v, rows_v, acc_sh, ident_v, sem):
        sid = lax.axis_index("s")
        wid = sid * info.num_cores + lax.axis_index("c")
        pltpu.sync_copy(ident_hbm, ident_v)
        for t in range(bpw):
            b = wid * bpw + t
            pltpu.sync_copy(gidx_hbm.at[b], idx_v)
            cps = []
            for kk in range(K):
                cp = pltpu.make_async_copy(
                    table_hbm.at[idx_v.at[kk]], rows_v.at[kk], sem)
                cp.start()
                cps.append(cp)
            for cp in cps:
                cp.wait()
            pltpu.sync_copy(rows_v.at[0], acc_sh.at[sid].at[ident_v])
            adds = []
            for kk in range(1, K):
                adds.append(pltpu.async_copy(
                    rows_v.at[kk], acc_sh.at[sid].at[ident_v], sem, add=True))
            for cp in adds:
                cp.wait()
            pltpu.sync_copy(acc_sh.at[sid], out_hbm.at[pl.ds(b * N, N)])

    return sc_gather


def _tc_full_body(edges_ref, atoms_ref, w1_ref, w2_ref, aux_ref, sel_ref,
                  iota_ref, summed_ref, out1_ref, out2_ref):
    bb, n, k = edges_ref.shape
    d = atoms_ref.shape[-1]
    kcw = w1_ref.shape[-1]
    cw = kcw // k
    iota_f = iota_ref[...]
    ones_1n = jnp.ones((1, n), jnp.float32)
    b1 = aux_ref[0:1, :]
    b2 = aux_ref[1:2, :]
    jdiv = aux_ref[2:3, :]
    for i in range(bb):
        e_f = edges_ref[i]
        a_cnt = jnp.zeros((n, n), jnp.float32)
        for j in range(k):
            col = jax.lax.slice(e_f, (0, j), (n, j + 1))
            col_b = jnp.dot(col, ones_1n,
                            preferred_element_type=jnp.float32)
            a_cnt += (col_b == iota_f).astype(jnp.float32)
        summed_ref[i] = jnp.dot(a_cnt, atoms_ref[i],
                                preferred_element_type=jnp.float32)
    e_all = edges_ref[...].reshape(bb * n, k)
    mask_e = (e_all != -1.0).astype(jnp.float32)
    ones_kj = jnp.ones((k, kcw), jnp.float32)
    deg_b = jnp.dot(mask_e, ones_kj,
                    preferred_element_type=jnp.float32)
    m_sel = (deg_b == jdiv).astype(jnp.float32)
    s_all = summed_ref[...].reshape(bb * n, d)
    a_all = atoms_ref[...].reshape(bb * n, d)
    z1 = jnp.maximum(jnp.dot(s_all, w1_ref[...],
                             preferred_element_type=jnp.float32) + b1, 0.0)
    z2 = jnp.maximum(jnp.dot(a_all, w2_ref[...],
                             preferred_element_type=jnp.float32) + b2, 0.0)
    o1 = jnp.dot(z1 * m_sel, sel_ref[...],
                 preferred_element_type=jnp.float32)
    o2 = jnp.dot(z2 * m_sel, sel_ref[...],
                 preferred_element_type=jnp.float32)
    out1_ref[...] = o1.reshape(bb, n, cw)
    out2_ref[...] = o2.reshape(bb, n, cw)


def _tc_dense_body(edges_ref, atoms_ref, summed_ref, w1_ref, w2_ref, aux_ref,
                   sel_ref, dep_ref, out1_ref, out2_ref):
    del dep_ref
    bb, n, k = edges_ref.shape
    d = atoms_ref.shape[-1]
    kcw = w1_ref.shape[-1]
    cw = kcw // k
    b1 = aux_ref[0:1, :]
    b2 = aux_ref[1:2, :]
    jdiv = aux_ref[2:3, :]
    e_all = edges_ref[...].reshape(bb * n, k)
    mask_e = (e_all != -1.0).astype(jnp.float32)
    ones_kj = jnp.ones((k, kcw), jnp.float32)
    deg_b = jnp.dot(mask_e, ones_kj,
                    preferred_element_type=jnp.float32)
    m_sel = (deg_b == jdiv).astype(jnp.float32)
    s_all = summed_ref[...].reshape(bb * n, d)
    a_all = atoms_ref[...].reshape(bb * n, d)
    z1 = jnp.maximum(jnp.dot(s_all, w1_ref[...],
                             preferred_element_type=jnp.float32) + b1, 0.0)
    z2 = jnp.maximum(jnp.dot(a_all, w2_ref[...],
                             preferred_element_type=jnp.float32) + b2, 0.0)
    o1 = jnp.dot(z1 * m_sel, sel_ref[...],
                 preferred_element_type=jnp.float32)
    o2 = jnp.dot(z2 * m_sel, sel_ref[...],
                 preferred_element_type=jnp.float32)
    out1_ref[...] = o1.reshape(bb, n, cw)
    out2_ref[...] = o2.reshape(bb, n, cw)


def kernel(atoms, edges, W1, b1, W2, b2):
    B, N, D = atoms.shape
    K = edges.shape[-1]
    CW = W1.shape[-1]
    S = SC_B
    gidx = (edges[:S].transpose(0, 2, 1)
            + (jnp.arange(S, dtype=jnp.int32) * N)[:, None, None])
    table = atoms.reshape(B * N, D)
    ident = jnp.arange(N, dtype=jnp.int32)
    summed_lo = _make_sc_gather(S, B, N, D, K)(gidx, table, ident)
    summed_lo = summed_lo.reshape(S, N, D)
    w1r = W1.transpose(1, 0, 2).reshape(D, K * CW)
    w2r = W2.transpose(1, 0, 2).reshape(D, K * CW)
    aux = jnp.zeros((8, K * CW), jnp.float32)
    aux = aux.at[0].set(b1.reshape(-1)).at[1].set(b2.reshape(-1))
    aux = aux.at[2].set(jnp.repeat(jnp.arange(K, dtype=jnp.float32), CW))
    sel = jnp.tile(jnp.eye(CW, dtype=jnp.float32), (K, 1))
    edges_f = edges.astype(jnp.float32)
    iota_f = jnp.broadcast_to(jnp.arange(N, dtype=jnp.float32)[None, :],
                              (N, N))
    sb = S // BB
    summed_hi, o1_hi, o2_hi = pl.pallas_call(
        _tc_full_body,
        grid=((B - S) // BB,),
        in_specs=[
            pl.BlockSpec((BB, N, K), lambda i: (i + sb, 0, 0)),
            pl.BlockSpec((BB, N, D), lambda i: (i + sb, 0, 0)),
            pl.BlockSpec((D, K * CW), lambda i: (0, 0)),
            pl.BlockSpec((D, K * CW), lambda i: (0, 0)),
            pl.BlockSpec((8, K * CW), lambda i: (0, 0)),
            pl.BlockSpec((K * CW, CW), lambda i: (0, 0)),
            pl.BlockSpec((N, N), lambda i: (0, 0)),
        ],
        out_specs=[
            pl.BlockSpec((BB, N, D), lambda i: (i, 0, 0)),
            pl.BlockSpec((BB, N, CW), lambda i: (i, 0, 0)),
            pl.BlockSpec((BB, N, CW), lambda i: (i, 0, 0)),
        ],
        out_shape=[
            jax.ShapeDtypeStruct((B - S, N, D), jnp.float32),
            jax.ShapeDtypeStruct((B - S, N, CW), jnp.float32),
            jax.ShapeDtypeStruct((B - S, N, CW), jnp.float32),
        ],
    )(edges_f, atoms, w1r, w2r, aux, sel, iota_f)
    o1_lo, o2_lo = pl.pallas_call(
        _tc_dense_body,
        grid=(S // BB,),
        in_specs=[
            pl.BlockSpec((BB, N, K), lambda i: (i, 0, 0)),
            pl.BlockSpec((BB, N, D), lambda i: (i, 0, 0)),
            pl.BlockSpec((BB, N, D), lambda i: (i, 0, 0)),
            pl.BlockSpec((D, K * CW), lambda i: (0, 0)),
            pl.BlockSpec((D, K * CW), lambda i: (0, 0)),
            pl.BlockSpec((8, K * CW), lambda i: (0, 0)),
            pl.BlockSpec((K * CW, CW), lambda i: (0, 0)),
            pl.BlockSpec((1, N, CW), lambda i: (0, 0, 0)),
        ],
        out_specs=[
            pl.BlockSpec((BB, N, CW), lambda i: (i, 0, 0)),
            pl.BlockSpec((BB, N, CW), lambda i: (i, 0, 0)),
        ],
        out_shape=[
            jax.ShapeDtypeStruct((S, N, CW), jnp.float32),
            jax.ShapeDtypeStruct((S, N, CW), jnp.float32),
        ],
    )(edges_f, atoms, summed_lo, w1r, w2r, aux, sel, o1_hi)
    summed = jnp.concatenate([summed_lo, summed_hi], axis=0)
    o1 = jnp.concatenate([o1_lo, o1_hi], axis=0)
    o2 = jnp.concatenate([o2_lo, o2_hi], axis=0)
    return (summed, atoms, o1, o2)
